# baseline (speedup 1.0000x reference)
import functools

import jax
import jax.numpy as jnp
from jax import lax
from jax.experimental import pallas as pl
from jax.experimental.pallas import tpu as pltpu
from jax.experimental.pallas import tpu_sc as plsc

NC = 2
NS = 16
NW = NC * NS
K = 128
FRAC0 = 0.74


def _mesh():
    return plsc.VectorSubcoreMesh(core_axis_name="c", subcore_axis_name="s")


def _deg_kernel(nacc, cpt0, cpt1, rpt):
    cpt_max = max(cpt0, cpt1)

    @functools.partial(
        pl.kernel,
        out_type=jax.ShapeDtypeStruct((NC, nacc), jnp.float32),
        mesh=_mesh(),
        scratch_types=[
            pltpu.VMEM((cpt_max, 1, K), jnp.int32),
            pltpu.VMEM((K,), jnp.float32),
            pltpu.VMEM_SHARED((nacc,), jnp.float32),
        ],
    )
    def deg(dst_hbm, ones_hbm, zer_hbm, out_hbm, dst_v, ones_v, deg_sh):
        cid = lax.axis_index("c")
        sid = lax.axis_index("s")
        nch = jnp.where(cid == 0, cpt0, cpt1)
        c0 = jnp.where(cid == 0, sid * cpt0, NS * cpt0 + sid * cpt1)
        r0 = sid * rpt

        @pl.when(cid == 0)
        def _():
            pltpu.sync_copy(dst_hbm.at[pl.ds(sid * cpt0, cpt0)],
                            dst_v.at[pl.ds(0, cpt0)])

        @pl.when(cid == 1)
        def _():
            pltpu.sync_copy(dst_hbm.at[pl.ds(NS * cpt0 + sid * cpt1, cpt1)],
                            dst_v.at[pl.ds(0, cpt1)])

        pltpu.sync_copy(ones_hbm, ones_v)
        pltpu.sync_copy(zer_hbm.at[pl.ds(r0, rpt)], deg_sh.at[pl.ds(r0, rpt)])
        plsc.subcore_barrier()

        def body(j, carry):
            pltpu.sync_copy(ones_v, deg_sh.at[dst_v.at[j, 0]], add=True)
            return carry

        lax.fori_loop(0, nch, body, 0)
        plsc.subcore_barrier()
        pltpu.sync_copy(deg_sh.at[pl.ds(r0, rpt)],
                        out_hbm.at[cid, pl.ds(r0, rpt)])

    return deg


def _edge_kernel(nacc, cpt0, cpt1, rpt, F, tc_tiling=True):
    cpt_max = max(cpt0, cpt1)

    @functools.partial(
        pl.kernel,
        out_type=jax.ShapeDtypeStruct((NC, nacc, F), jnp.float32),
        mesh=_mesh(),
        compiler_params=pltpu.CompilerParams(use_tc_tiling_on_sc=tc_tiling),
        scratch_types=[
            pltpu.VMEM((1, K), jnp.int32),
            pltpu.VMEM((1, K), jnp.int32),
            pltpu.VMEM((cpt_max, 1, K), jnp.int32),
            pltpu.VMEM((K, F), jnp.float32),
            pltpu.VMEM((K, F), jnp.float32),
            pltpu.VMEM_SHARED((nacc, F), jnp.float32),
            pltpu.SemaphoreType.DMA,
            pltpu.SemaphoreType.DMA,
            pltpu.SemaphoreType.DMA,
        ],
    )
    def edge(tab_hbm, src_hbm, dst_hbm, zer_hbm, out_hbm,
             sia, sib, dst_v, rows_a, rows_b, acc_sh, gsem, ssem, isem):
        cid = lax.axis_index("c")
        sid = lax.axis_index("s")
        nch = jnp.where(cid == 0, cpt0, cpt1)
        c0 = jnp.where(cid == 0, sid * cpt0, NS * cpt0 + sid * cpt1)
        r0 = sid * rpt

        @pl.when(cid == 0)
        def _():
            pltpu.sync_copy(dst_hbm.at[pl.ds(sid * cpt0, cpt0)],
                            dst_v.at[pl.ds(0, cpt0)])

        @pl.when(cid == 1)
        def _():
            pltpu.sync_copy(dst_hbm.at[pl.ds(NS * cpt0 + sid * cpt1, cpt1)],
                            dst_v.at[pl.ds(0, cpt1)])

        pltpu.sync_copy(src_hbm.at[c0], sia)
        pltpu.async_copy(tab_hbm.at[sia.at[0]], rows_a, gsem)

        @pl.when(nch > 1)
        def _():
            pltpu.async_copy(src_hbm.at[c0 + 1], sib, isem)

        pltpu.sync_copy(zer_hbm.at[pl.ds(r0, rpt)], acc_sh.at[pl.ds(r0, rpt)])
        plsc.subcore_barrier()

        def do_iter(j, s_cur, s_nxt, rows_cur, rows_nxt):
            @pl.when(j >= 1)
            def _():
                pltpu.make_async_copy(
                    rows_nxt, acc_sh.at[dst_v.at[j - 1, 0]], ssem).wait()

            @pl.when(j + 1 < nch)
            def _():
                pltpu.make_async_copy(
                    src_hbm.at[c0 + j + 1], s_nxt, isem).wait()
                pltpu.async_copy(tab_hbm.at[s_nxt.at[0]], rows_nxt, gsem)

            pltpu.make_async_copy(
                tab_hbm.at[s_cur.at[0]], rows_cur, gsem).wait()

            @pl.when(j + 2 < nch)
            def _():
                pltpu.async_copy(src_hbm.at[c0 + j + 2], s_cur, isem)

            pltpu.async_copy(rows_cur, acc_sh.at[dst_v.at[j, 0]], ssem, add=True)

        def body(j, carry):
            par = lax.rem(j, 2)

            @pl.when(par == 0)
            def _():
                do_iter(j, sia, sib, rows_a, rows_b)

            @pl.when(par == 1)
            def _():
                do_iter(j, sib, sia, rows_b, rows_a)

            return carry

        lax.fori_loop(0, nch, body, 0)
        last = nch - 1

        @pl.when(lax.rem(last, 2) == 0)
        def _():
            pltpu.make_async_copy(rows_a, acc_sh.at[dst_v.at[last, 0]], ssem).wait()

        @pl.when(lax.rem(last, 2) == 1)
        def _():
            pltpu.make_async_copy(rows_b, acc_sh.at[dst_v.at[last, 0]], ssem).wait()

        plsc.subcore_barrier()
        pltpu.sync_copy(acc_sh.at[pl.ds(r0, rpt)],
                        out_hbm.at[cid, pl.ds(r0, rpt)])

    return edge


def _pick_bn(n):
    for bn in (2000, 1024, 512, 400, 256, 200, 128, 80, 64, 40, 32, 16, 8):
        if n % bn == 0:
            return bn
    return 1


def _flat_chunks(arr, pad_val, tot_ch, n_real):
    flat = jnp.concatenate(
        [arr, jnp.full((tot_ch * K - n_real,), pad_val, jnp.int32)])
    return flat.reshape(tot_ch, 1, K)


def kernel(x, edge_index, W1, b1, W2, b2):
    N, D = x.shape
    E = edge_index.shape[1]
    C = W2.shape[1]
    CP = 16

    nch = -(-E // K)
    cpt0 = max(1, round(nch * FRAC0 / NS))
    cpt1 = -(-(nch - cpt0 * NS) // NS)
    cpt0s = max(1, round(nch * 0.5 / NS))
    cpt1s = -(-(nch - cpt0s * NS) // NS)
    tot_ch = max((cpt0 + cpt1) * NS, (cpt0s + cpt1s) * NS)
    nacc = -(-(N + 1) // 128) * 128
    rpt = nacc // NS
    nacc_d = -(-(N + 1) // (NS * 256)) * (NS * 256)
    rpt_d = nacc_d // NS
    BN = _pick_bn(N)

    f32 = jnp.float32
    src_p = _flat_chunks(lax.optimization_barrier(edge_index)[0], 0, tot_ch, E)
    dst_p = _flat_chunks(edge_index[1], N, tot_ch, E)
    ones_k = jnp.ones((K,), f32)
    zer1 = jnp.zeros((nacc_d,), f32)
    zerD = jnp.zeros((nacc, D), f32)
    zerC = jnp.zeros((nacc, CP), f32)
    b1r = b1.reshape(1, D)
    W2p = jnp.zeros((D, CP), f32).at[:, :C].set(W2)
    b2p = jnp.zeros((1, CP), f32).at[0, :C].set(b2)

    deg_part = _deg_kernel(nacc_d, cpt0s, cpt1s, rpt_d)(dst_p, ones_k, zer1)
    deg_t = deg_part.T[:N]

    def mm1a_body(x_ref, w_ref, h_ref):
        h_ref[...] = jnp.dot(x_ref[...], w_ref[...], preferred_element_type=f32)

    h = pl.pallas_call(
        mm1a_body,
        grid=(N // BN,),
        in_specs=[
            pl.BlockSpec((BN, D), lambda i: (i, 0)),
            pl.BlockSpec((D, D), lambda i: (0, 0)),
        ],
        out_specs=pl.BlockSpec((BN, D), lambda i: (i, 0)),
        out_shape=jax.ShapeDtypeStruct((N, D), f32),
    )(x, W1)

    def mm1b_body(h_ref, dg_ref, hs_ref, dv_ref):
        dv = lax.rsqrt(dg_ref[:, 0:1] + dg_ref[:, 1:2] + 1.0)
        hs_ref[...] = h_ref[...] * dv
        dv_ref[...] = dv

    hs, dinv = pl.pallas_call(
        mm1b_body,
        grid=(N // BN,),
        in_specs=[
            pl.BlockSpec((BN, D), lambda i: (i, 0)),
            pl.BlockSpec((BN, 2), lambda i: (i, 0)),
        ],
        out_specs=[
            pl.BlockSpec((BN, D), lambda i: (i, 0)),
            pl.BlockSpec((BN, 1), lambda i: (i, 0)),
        ],
        out_shape=[
            jax.ShapeDtypeStruct((N, D), f32),
            jax.ShapeDtypeStruct((N, 1), f32),
        ],
    )(h, deg_t)

    acc1 = _edge_kernel(nacc, cpt0, cpt1, rpt, D)(hs, src_p, dst_p, zerD)

    def mid_body(a_ref, hs_ref, dv_ref, b1_ref, w2_ref, out_ref):
        t = (a_ref[0] + a_ref[1] + hs_ref[...]) * dv_ref[...] + b1_ref[...]
        g = jnp.where(t >= 0, t, 0.01 * t)
        h2 = jnp.dot(g, w2_ref[...], preferred_element_type=f32)
        out_ref[...] = h2 * dv_ref[...]

    hs2 = pl.pallas_call(
        mid_body,
        grid=(N // BN,),
        in_specs=[
            pl.BlockSpec((NC, BN, D), lambda i: (0, i, 0)),
            pl.BlockSpec((BN, D), lambda i: (i, 0)),
            pl.BlockSpec((BN, 1), lambda i: (i, 0)),
            pl.BlockSpec((1, D), lambda i: (0, 0)),
            pl.BlockSpec((D, CP), lambda i: (0, 0)),
        ],
        out_specs=pl.BlockSpec((BN, CP), lambda i: (i, 0)),
        out_shape=jax.ShapeDtypeStruct((N, CP), f32),
    )(acc1, hs, dinv, b1r, W2p)

    acc2 = _edge_kernel(nacc, cpt0s, cpt1s, rpt, CP, tc_tiling=False)(
        hs2, src_p, dst_p, zerC)

    def fin_body(a_ref, hs_ref, dv_ref, b2_ref, out_ref):
        out_ref[...] = (a_ref[0] + a_ref[1] + hs_ref[...]) * dv_ref[...] + b2_ref[...]

    outp = pl.pallas_call(
        fin_body,
        grid=(N // BN,),
        in_specs=[
            pl.BlockSpec((NC, BN, CP), lambda i: (0, i, 0)),
            pl.BlockSpec((BN, CP), lambda i: (i, 0)),
            pl.BlockSpec((BN, 1), lambda i: (i, 0)),
            pl.BlockSpec((1, CP), lambda i: (0, 0)),
        ],
        out_specs=pl.BlockSpec((BN, CP), lambda i: (i, 0)),
        out_shape=jax.ShapeDtypeStruct((N, CP), f32),
    )(acc2, hs2, dinv, b2p)

    return outp[:, :C]


# speedup vs baseline: 1.0170x; 1.0170x over previous
import functools

import jax
import jax.numpy as jnp
from jax import lax
from jax.experimental import pallas as pl
from jax.experimental.pallas import tpu as pltpu
from jax.experimental.pallas import tpu_sc as plsc

NC = 2
NS = 16
NW = NC * NS
K = 128
FRAC0 = 0.74


def _mesh():
    return plsc.VectorSubcoreMesh(core_axis_name="c", subcore_axis_name="s")


def _deg_kernel(nacc, cpt0, cpt1, rpt):
    cpt_max = max(cpt0, cpt1)

    @functools.partial(
        pl.kernel,
        out_type=jax.ShapeDtypeStruct((NC, nacc), jnp.float32),
        mesh=_mesh(),
        scratch_types=[
            pltpu.VMEM((cpt_max, 1, K), jnp.int32),
            pltpu.VMEM((K,), jnp.float32),
            pltpu.VMEM_SHARED((nacc,), jnp.float32),
        ],
    )
    def deg(dst_hbm, ones_hbm, zer_hbm, out_hbm, dst_v, ones_v, deg_sh):
        cid = lax.axis_index("c")
        sid = lax.axis_index("s")
        nch = jnp.where(cid == 0, cpt0, cpt1)
        c0 = jnp.where(cid == 0, sid * cpt0, NS * cpt0 + sid * cpt1)
        r0 = sid * rpt

        @pl.when(cid == 0)
        def _():
            pltpu.sync_copy(dst_hbm.at[pl.ds(sid * cpt0, cpt0)],
                            dst_v.at[pl.ds(0, cpt0)])

        @pl.when(cid == 1)
        def _():
            pltpu.sync_copy(dst_hbm.at[pl.ds(NS * cpt0 + sid * cpt1, cpt1)],
                            dst_v.at[pl.ds(0, cpt1)])

        pltpu.sync_copy(ones_hbm, ones_v)
        pltpu.sync_copy(zer_hbm.at[pl.ds(r0, rpt)], deg_sh.at[pl.ds(r0, rpt)])
        plsc.subcore_barrier()

        def body(j, carry):
            pltpu.sync_copy(ones_v, deg_sh.at[dst_v.at[j, 0]], add=True)
            return carry

        lax.fori_loop(0, nch, body, 0)
        plsc.subcore_barrier()
        pltpu.sync_copy(deg_sh.at[pl.ds(r0, rpt)],
                        out_hbm.at[cid, pl.ds(r0, rpt)])

    return deg


def _edge_kernel(nacc, cpt0, cpt1, rpt, F, tc_tiling=True):
    cpt_max = max(cpt0, cpt1)

    @functools.partial(
        pl.kernel,
        out_type=jax.ShapeDtypeStruct((NC, nacc, F), jnp.float32),
        mesh=_mesh(),
        compiler_params=pltpu.CompilerParams(use_tc_tiling_on_sc=tc_tiling),
        scratch_types=[
            pltpu.VMEM((1, K), jnp.int32),
            pltpu.VMEM((1, K), jnp.int32),
            pltpu.VMEM((cpt_max, 1, K), jnp.int32),
            pltpu.VMEM((K, F), jnp.float32),
            pltpu.VMEM((K, F), jnp.float32),
            pltpu.VMEM_SHARED((nacc, F), jnp.float32),
            pltpu.SemaphoreType.DMA,
            pltpu.SemaphoreType.DMA,
            pltpu.SemaphoreType.DMA,
        ],
    )
    def edge(tab_hbm, src_hbm, dst_hbm, zer_hbm, out_hbm,
             sia, sib, dst_v, rows_a, rows_b, acc_sh, gsem, ssem, isem):
        cid = lax.axis_index("c")
        sid = lax.axis_index("s")
        nch = jnp.where(cid == 0, cpt0, cpt1)
        c0 = jnp.where(cid == 0, sid * cpt0, NS * cpt0 + sid * cpt1)
        r0 = sid * rpt

        @pl.when(cid == 0)
        def _():
            pltpu.sync_copy(dst_hbm.at[pl.ds(sid * cpt0, cpt0)],
                            dst_v.at[pl.ds(0, cpt0)])

        @pl.when(cid == 1)
        def _():
            pltpu.sync_copy(dst_hbm.at[pl.ds(NS * cpt0 + sid * cpt1, cpt1)],
                            dst_v.at[pl.ds(0, cpt1)])

        pltpu.sync_copy(src_hbm.at[c0], sia)
        pltpu.async_copy(tab_hbm.at[sia.at[0]], rows_a, gsem)

        @pl.when(nch > 1)
        def _():
            pltpu.async_copy(src_hbm.at[c0 + 1], sib, isem)

        pltpu.sync_copy(zer_hbm.at[pl.ds(r0, rpt)], acc_sh.at[pl.ds(r0, rpt)])
        plsc.subcore_barrier()

        def do_iter(j, s_cur, s_nxt, rows_cur, rows_nxt):
            @pl.when(j >= 1)
            def _():
                pltpu.make_async_copy(
                    rows_nxt, acc_sh.at[dst_v.at[j - 1, 0]], ssem).wait()

            @pl.when(j + 1 < nch)
            def _():
                pltpu.make_async_copy(
                    src_hbm.at[c0 + j + 1], s_nxt, isem).wait()
                pltpu.async_copy(tab_hbm.at[s_nxt.at[0]], rows_nxt, gsem)

            pltpu.make_async_copy(
                tab_hbm.at[s_cur.at[0]], rows_cur, gsem).wait()

            @pl.when(j + 2 < nch)
            def _():
                pltpu.async_copy(src_hbm.at[c0 + j + 2], s_cur, isem)

            pltpu.async_copy(rows_cur, acc_sh.at[dst_v.at[j, 0]], ssem, add=True)

        def body(j, carry):
            par = lax.rem(j, 2)

            @pl.when(par == 0)
            def _():
                do_iter(j, sia, sib, rows_a, rows_b)

            @pl.when(par == 1)
            def _():
                do_iter(j, sib, sia, rows_b, rows_a)

            return carry

        lax.fori_loop(0, nch, body, 0)
        last = nch - 1

        @pl.when(lax.rem(last, 2) == 0)
        def _():
            pltpu.make_async_copy(rows_a, acc_sh.at[dst_v.at[last, 0]], ssem).wait()

        @pl.when(lax.rem(last, 2) == 1)
        def _():
            pltpu.make_async_copy(rows_b, acc_sh.at[dst_v.at[last, 0]], ssem).wait()

        plsc.subcore_barrier()
        pltpu.sync_copy(acc_sh.at[pl.ds(r0, rpt)],
                        out_hbm.at[cid, pl.ds(r0, rpt)])

    return edge


def _pick_bn(n):
    for bn in (2000, 1024, 512, 400, 256, 200, 128, 80, 64, 40, 32, 16, 8):
        if n % bn == 0:
            return bn
    return 1


def _flat_chunks(arr, pad_val, tot_ch, n_real):
    flat = jnp.concatenate(
        [arr, jnp.full((tot_ch * K - n_real,), pad_val, jnp.int32)])
    return flat.reshape(tot_ch, 1, K)


def kernel(x, edge_index, W1, b1, W2, b2):
    N, D = x.shape
    E = edge_index.shape[1]
    C = W2.shape[1]
    CP = 16

    nch = -(-E // K)
    cpt0 = max(1, round(nch * FRAC0 / NS))
    cpt1 = -(-(nch - cpt0 * NS) // NS)
    cpt0s = max(1, round(nch * 0.5 / NS))
    cpt1s = -(-(nch - cpt0s * NS) // NS)
    tot_ch = max((cpt0 + cpt1) * NS, (cpt0s + cpt1s) * NS)
    nacc = -(-(N + 1) // 128) * 128
    rpt = nacc // NS
    nacc_d = -(-(N + 1) // (NS * 256)) * (NS * 256)
    rpt_d = nacc_d // NS
    BN = _pick_bn(N)

    f32 = jnp.float32
    src_p = _flat_chunks(edge_index[0], 0, tot_ch, E)
    dst_p = _flat_chunks(edge_index[1], N, tot_ch, E)
    ones_k = jnp.ones((K,), f32)
    zer1 = jnp.zeros((nacc_d,), f32)
    zerD = jnp.zeros((nacc, D), f32)
    zerC = jnp.zeros((nacc, CP), f32)
    b1r = b1.reshape(1, D)
    W2p = jnp.zeros((D, CP), f32).at[:, :C].set(W2)
    b2p = jnp.zeros((1, CP), f32).at[0, :C].set(b2)

    deg_part = _deg_kernel(nacc_d, cpt0s, cpt1s, rpt_d)(dst_p, ones_k, zer1)
    deg_t = deg_part.T[:N]

    def mm1a_body(x_ref, w_ref, h_ref):
        h_ref[...] = jnp.dot(x_ref[...], w_ref[...], preferred_element_type=f32)

    h = pl.pallas_call(
        mm1a_body,
        grid=(N // BN,),
        in_specs=[
            pl.BlockSpec((BN, D), lambda i: (i, 0)),
            pl.BlockSpec((D, D), lambda i: (0, 0)),
        ],
        out_specs=pl.BlockSpec((BN, D), lambda i: (i, 0)),
        out_shape=jax.ShapeDtypeStruct((N, D), f32),
    )(x, W1)

    def mm1b_body(h_ref, dg_ref, hs_ref, dv_ref):
        dv = lax.rsqrt(dg_ref[:, 0:1] + dg_ref[:, 1:2] + 1.0)
        hs_ref[...] = h_ref[...] * dv
        dv_ref[...] = dv

    hs, dinv = pl.pallas_call(
        mm1b_body,
        grid=(N // BN,),
        in_specs=[
            pl.BlockSpec((BN, D), lambda i: (i, 0)),
            pl.BlockSpec((BN, 2), lambda i: (i, 0)),
        ],
        out_specs=[
            pl.BlockSpec((BN, D), lambda i: (i, 0)),
            pl.BlockSpec((BN, 1), lambda i: (i, 0)),
        ],
        out_shape=[
            jax.ShapeDtypeStruct((N, D), f32),
            jax.ShapeDtypeStruct((N, 1), f32),
        ],
    )(h, deg_t)

    acc1 = _edge_kernel(nacc, cpt0, cpt1, rpt, D)(hs, src_p, dst_p, zerD)

    def mid_body(a_ref, hs_ref, dv_ref, b1_ref, w2_ref, out_ref):
        t = (a_ref[0] + a_ref[1] + hs_ref[...]) * dv_ref[...] + b1_ref[...]
        g = jnp.where(t >= 0, t, 0.01 * t)
        h2 = jnp.dot(g, w2_ref[...], preferred_element_type=f32)
        out_ref[...] = h2 * dv_ref[...]

    hs2 = pl.pallas_call(
        mid_body,
        grid=(N // BN,),
        in_specs=[
            pl.BlockSpec((NC, BN, D), lambda i: (0, i, 0)),
            pl.BlockSpec((BN, D), lambda i: (i, 0)),
            pl.BlockSpec((BN, 1), lambda i: (i, 0)),
            pl.BlockSpec((1, D), lambda i: (0, 0)),
            pl.BlockSpec((D, CP), lambda i: (0, 0)),
        ],
        out_specs=pl.BlockSpec((BN, CP), lambda i: (i, 0)),
        out_shape=jax.ShapeDtypeStruct((N, CP), f32),
    )(acc1, hs, dinv, b1r, W2p)

    acc2 = _edge_kernel(nacc, cpt0s, cpt1s, rpt, CP, tc_tiling=False)(
        hs2, src_p, dst_p, zerC)

    def fin_body(a_ref, hs_ref, dv_ref, b2_ref, out_ref):
        out_ref[...] = (a_ref[0] + a_ref[1] + hs_ref[...]) * dv_ref[...] + b2_ref[...]

    outp = pl.pallas_call(
        fin_body,
        grid=(N // BN,),
        in_specs=[
            pl.BlockSpec((NC, BN, CP), lambda i: (0, i, 0)),
            pl.BlockSpec((BN, CP), lambda i: (i, 0)),
            pl.BlockSpec((BN, 1), lambda i: (i, 0)),
            pl.BlockSpec((1, CP), lambda i: (0, 0)),
        ],
        out_specs=pl.BlockSpec((BN, CP), lambda i: (i, 0)),
        out_shape=jax.ShapeDtypeStruct((N, CP), f32),
    )(acc2, hs2, dinv, b2p)

    return outp[:, :C]


# speedup vs baseline: 1.0601x; 1.0424x over previous
import functools

import jax
import jax.numpy as jnp
from jax import lax
from jax.experimental import pallas as pl
from jax.experimental.pallas import tpu as pltpu
from jax.experimental.pallas import tpu_sc as plsc

NC = 2
NS = 16
NW = NC * NS
K = 128
FRAC0 = 0.72


def _mesh():
    return plsc.VectorSubcoreMesh(core_axis_name="c", subcore_axis_name="s")


def _deg_kernel(nacc, cpt0, cpt1, rpt):
    cpt_max = max(cpt0, cpt1)

    @functools.partial(
        pl.kernel,
        out_type=jax.ShapeDtypeStruct((NC, nacc), jnp.float32),
        mesh=_mesh(),
        scratch_types=[
            pltpu.VMEM((cpt_max, 1, K), jnp.int32),
            pltpu.VMEM((K,), jnp.float32),
            pltpu.VMEM_SHARED((nacc,), jnp.float32),
        ],
    )
    def deg(dst_hbm, ones_hbm, zer_hbm, out_hbm, dst_v, ones_v, deg_sh):
        cid = lax.axis_index("c")
        sid = lax.axis_index("s")
        nch = jnp.where(cid == 0, cpt0, cpt1)
        c0 = jnp.where(cid == 0, sid * cpt0, NS * cpt0 + sid * cpt1)
        r0 = sid * rpt

        @pl.when(cid == 0)
        def _():
            pltpu.sync_copy(dst_hbm.at[pl.ds(sid * cpt0, cpt0)],
                            dst_v.at[pl.ds(0, cpt0)])

        @pl.when(cid == 1)
        def _():
            pltpu.sync_copy(dst_hbm.at[pl.ds(NS * cpt0 + sid * cpt1, cpt1)],
                            dst_v.at[pl.ds(0, cpt1)])

        pltpu.sync_copy(ones_hbm, ones_v)
        pltpu.sync_copy(zer_hbm.at[pl.ds(r0, rpt)], deg_sh.at[pl.ds(r0, rpt)])
        plsc.subcore_barrier()

        def body(j, carry):
            pltpu.sync_copy(ones_v, deg_sh.at[dst_v.at[j, 0]], add=True)
            return carry

        lax.fori_loop(0, nch, body, 0)
        plsc.subcore_barrier()
        pltpu.sync_copy(deg_sh.at[pl.ds(r0, rpt)],
                        out_hbm.at[cid, pl.ds(r0, rpt)])

    return deg


def _edge_kernel(nacc, cpt0, cpt1, rpt, F, tc_tiling=True):
    cpt_max = max(cpt0, cpt1)

    @functools.partial(
        pl.kernel,
        out_type=jax.ShapeDtypeStruct((NC, nacc, F), jnp.float32),
        mesh=_mesh(),
        compiler_params=pltpu.CompilerParams(use_tc_tiling_on_sc=tc_tiling),
        scratch_types=[
            pltpu.VMEM((1, K), jnp.int32),
            pltpu.VMEM((1, K), jnp.int32),
            pltpu.VMEM((cpt_max, 1, K), jnp.int32),
            pltpu.VMEM((K, F), jnp.float32),
            pltpu.VMEM((K, F), jnp.float32),
            pltpu.VMEM_SHARED((nacc, F), jnp.float32),
            pltpu.SemaphoreType.DMA,
            pltpu.SemaphoreType.DMA,
            pltpu.SemaphoreType.DMA,
        ],
    )
    def edge(tab_hbm, src_hbm, dst_hbm, zer_hbm, out_hbm,
             sia, sib, dst_v, rows_a, rows_b, acc_sh, gsem, ssem, isem):
        cid = lax.axis_index("c")
        sid = lax.axis_index("s")
        nch = jnp.where(cid == 0, cpt0, cpt1)
        c0 = jnp.where(cid == 0, sid * cpt0, NS * cpt0 + sid * cpt1)
        r0 = sid * rpt

        @pl.when(cid == 0)
        def _():
            pltpu.sync_copy(dst_hbm.at[pl.ds(sid * cpt0, cpt0)],
                            dst_v.at[pl.ds(0, cpt0)])

        @pl.when(cid == 1)
        def _():
            pltpu.sync_copy(dst_hbm.at[pl.ds(NS * cpt0 + sid * cpt1, cpt1)],
                            dst_v.at[pl.ds(0, cpt1)])

        pltpu.sync_copy(src_hbm.at[c0], sia)
        pltpu.async_copy(tab_hbm.at[sia.at[0]], rows_a, gsem)

        @pl.when(nch > 1)
        def _():
            pltpu.async_copy(src_hbm.at[c0 + 1], sib, isem)

        pltpu.sync_copy(zer_hbm.at[pl.ds(r0, rpt)], acc_sh.at[pl.ds(r0, rpt)])
        plsc.subcore_barrier()

        def do_iter(j, s_cur, s_nxt, rows_cur, rows_nxt):
            @pl.when(j >= 1)
            def _():
                pltpu.make_async_copy(
                    rows_nxt, acc_sh.at[dst_v.at[j - 1, 0]], ssem).wait()

            @pl.when(j + 1 < nch)
            def _():
                pltpu.make_async_copy(
                    src_hbm.at[c0 + j + 1], s_nxt, isem).wait()
                pltpu.async_copy(tab_hbm.at[s_nxt.at[0]], rows_nxt, gsem)

            pltpu.make_async_copy(
                tab_hbm.at[s_cur.at[0]], rows_cur, gsem).wait()

            @pl.when(j + 2 < nch)
            def _():
                pltpu.async_copy(src_hbm.at[c0 + j + 2], s_cur, isem)

            pltpu.async_copy(rows_cur, acc_sh.at[dst_v.at[j, 0]], ssem, add=True)

        def body(j, carry):
            par = lax.rem(j, 2)

            @pl.when(par == 0)
            def _():
                do_iter(j, sia, sib, rows_a, rows_b)

            @pl.when(par == 1)
            def _():
                do_iter(j, sib, sia, rows_b, rows_a)

            return carry

        lax.fori_loop(0, nch, body, 0)
        last = nch - 1

        @pl.when(lax.rem(last, 2) == 0)
        def _():
            pltpu.make_async_copy(rows_a, acc_sh.at[dst_v.at[last, 0]], ssem).wait()

        @pl.when(lax.rem(last, 2) == 1)
        def _():
            pltpu.make_async_copy(rows_b, acc_sh.at[dst_v.at[last, 0]], ssem).wait()

        plsc.subcore_barrier()
        pltpu.sync_copy(acc_sh.at[pl.ds(r0, rpt)],
                        out_hbm.at[cid, pl.ds(r0, rpt)])

    return edge


def _edge_kernel_deep(nacc, cpt0, cpt1, rpt, F, tc_tiling=True):
    cpt_max = max(cpt0, cpt1)
    DP = 4

    @functools.partial(
        pl.kernel,
        out_type=jax.ShapeDtypeStruct((NC, nacc, F), jnp.float32),
        mesh=_mesh(),
        compiler_params=pltpu.CompilerParams(use_tc_tiling_on_sc=tc_tiling),
        scratch_types=(
            [pltpu.VMEM((1, K), jnp.int32) for _ in range(DP)]
            + [pltpu.VMEM((K, F), jnp.float32) for _ in range(DP)]
            + [
                pltpu.VMEM((cpt_max, 1, K), jnp.int32),
                pltpu.VMEM_SHARED((nacc, F), jnp.float32),
                pltpu.SemaphoreType.DMA,
                pltpu.SemaphoreType.DMA,
                pltpu.SemaphoreType.DMA,
            ]
        ),
    )
    def edge(tab_hbm, src_hbm, dst_hbm, zer_hbm, out_hbm,
             si0, si1, si2, si3, ro0, ro1, ro2, ro3,
             dst_v, acc_sh, gsem, ssem, isem):
        sis = [si0, si1, si2, si3]
        ros = [ro0, ro1, ro2, ro3]
        cid = lax.axis_index("c")
        sid = lax.axis_index("s")
        nch = jnp.where(cid == 0, cpt0, cpt1)
        c0 = jnp.where(cid == 0, sid * cpt0, NS * cpt0 + sid * cpt1)
        r0 = sid * rpt

        @pl.when(cid == 0)
        def _():
            pltpu.sync_copy(dst_hbm.at[pl.ds(sid * cpt0, cpt0)],
                            dst_v.at[pl.ds(0, cpt0)])

        @pl.when(cid == 1)
        def _():
            pltpu.sync_copy(dst_hbm.at[pl.ds(NS * cpt0 + sid * cpt1, cpt1)],
                            dst_v.at[pl.ds(0, cpt1)])

        pltpu.sync_copy(src_hbm.at[c0], si0)
        pltpu.async_copy(tab_hbm.at[si0.at[0]], ro0, gsem)
        for t in (1, 2):
            @pl.when(nch > t)
            def _(t=t):
                pltpu.async_copy(src_hbm.at[c0 + t], sis[t], isem)
        pltpu.sync_copy(zer_hbm.at[pl.ds(r0, rpt)], acc_sh.at[pl.ds(r0, rpt)])
        plsc.subcore_barrier()

        def do_iter(j, p):
            s_n1, r_n1 = sis[(p + 1) % DP], ros[(p + 1) % DP]

            @pl.when(j >= 3)
            def _():
                pltpu.make_async_copy(
                    r_n1, acc_sh.at[dst_v.at[j - 3, 0]], ssem).wait()

            @pl.when(j + 1 < nch)
            def _():
                pltpu.make_async_copy(
                    src_hbm.at[c0 + j + 1], s_n1, isem).wait()
                pltpu.async_copy(tab_hbm.at[s_n1.at[0]], r_n1, gsem)

            @pl.when(j + 3 < nch)
            def _():
                pltpu.async_copy(src_hbm.at[c0 + j + 3], sis[(p + 3) % DP], isem)

            pltpu.make_async_copy(tab_hbm.at[sis[p].at[0]], ros[p], gsem).wait()
            pltpu.async_copy(ros[p], acc_sh.at[dst_v.at[j, 0]], ssem, add=True)

        def body(j, carry):
            par = lax.rem(j, DP)
            for p in range(DP):
                @pl.when(par == p)
                def _(p=p):
                    do_iter(j, p)
            return carry

        lax.fori_loop(0, nch, body, 0)

        def drain(t):
            for p in range(DP):
                @pl.when((t >= 0) & (lax.rem(t, DP) == p))
                def _(p=p):
                    pltpu.make_async_copy(
                        ros[p], acc_sh.at[dst_v.at[t, 0]], ssem).wait()

        drain(nch - 3)
        drain(nch - 2)
        drain(nch - 1)
        plsc.subcore_barrier()
        pltpu.sync_copy(acc_sh.at[pl.ds(r0, rpt)],
                        out_hbm.at[cid, pl.ds(r0, rpt)])

    return edge


def _pick_bn(n):
    for bn in (2000, 1024, 512, 400, 256, 200, 128, 80, 64, 40, 32, 16, 8):
        if n % bn == 0:
            return bn
    return 1


def _flat_chunks(arr, pad_val, tot_ch, n_real):
    flat = jnp.concatenate(
        [arr, jnp.full((tot_ch * K - n_real,), pad_val, jnp.int32)])
    return flat.reshape(tot_ch, 1, K)


def kernel(x, edge_index, W1, b1, W2, b2):
    N, D = x.shape
    E = edge_index.shape[1]
    C = W2.shape[1]
    CP = 16

    nch = -(-E // K)
    cpt0 = max(1, round(nch * FRAC0 / NS))
    cpt1 = -(-(nch - cpt0 * NS) // NS)
    cpt0s = max(1, round(nch * 0.5 / NS))
    cpt1s = -(-(nch - cpt0s * NS) // NS)
    tot_ch = max((cpt0 + cpt1) * NS, (cpt0s + cpt1s) * NS)
    nacc = -(-(N + 1) // 128) * 128
    rpt = nacc // NS
    nacc_d = -(-(N + 1) // (NS * 256)) * (NS * 256)
    rpt_d = nacc_d // NS
    BN = _pick_bn(N)

    f32 = jnp.float32
    src_p = _flat_chunks(edge_index[0], 0, tot_ch, E)
    dst_p = _flat_chunks(edge_index[1], N, tot_ch, E)
    ones_k = jnp.ones((K,), f32)
    zer1 = jnp.zeros((nacc_d,), f32)
    zerD = jnp.zeros((nacc, D), f32)
    zerC = jnp.zeros((nacc, CP), f32)
    b1r = b1.reshape(1, D)
    W2p = jnp.zeros((D, CP), f32).at[:, :C].set(W2)
    b2p = jnp.zeros((1, CP), f32).at[0, :C].set(b2)

    deg_part = _deg_kernel(nacc_d, cpt0s, cpt1s, rpt_d)(dst_p, ones_k, zer1)
    deg_t = deg_part.T[:N]

    def mm1a_body(x_ref, w_ref, h_ref):
        h_ref[...] = jnp.dot(x_ref[...], w_ref[...], preferred_element_type=f32)

    h = pl.pallas_call(
        mm1a_body,
        grid=(N // BN,),
        in_specs=[
            pl.BlockSpec((BN, D), lambda i: (i, 0)),
            pl.BlockSpec((D, D), lambda i: (0, 0)),
        ],
        out_specs=pl.BlockSpec((BN, D), lambda i: (i, 0)),
        out_shape=jax.ShapeDtypeStruct((N, D), f32),
    )(x, W1)

    def mm1b_body(h_ref, dg_ref, hs_ref, dv_ref):
        dv = lax.rsqrt(dg_ref[:, 0:1] + dg_ref[:, 1:2] + 1.0)
        hs_ref[...] = h_ref[...] * dv
        dv_ref[...] = dv

    hs, dinv = pl.pallas_call(
        mm1b_body,
        grid=(N // BN,),
        in_specs=[
            pl.BlockSpec((BN, D), lambda i: (i, 0)),
            pl.BlockSpec((BN, 2), lambda i: (i, 0)),
        ],
        out_specs=[
            pl.BlockSpec((BN, D), lambda i: (i, 0)),
            pl.BlockSpec((BN, 1), lambda i: (i, 0)),
        ],
        out_shape=[
            jax.ShapeDtypeStruct((N, D), f32),
            jax.ShapeDtypeStruct((N, 1), f32),
        ],
    )(h, deg_t)

    acc1 = _edge_kernel(nacc, cpt0, cpt1, rpt, D)(hs, src_p, dst_p, zerD)

    def mid_body(a_ref, hs_ref, dv_ref, b1_ref, w2_ref, out_ref):
        t = (a_ref[0] + a_ref[1] + hs_ref[...]) * dv_ref[...] + b1_ref[...]
        g = jnp.where(t >= 0, t, 0.01 * t)
        h2 = jnp.dot(g, w2_ref[...], preferred_element_type=f32)
        out_ref[...] = h2 * dv_ref[...]

    hs2 = pl.pallas_call(
        mid_body,
        grid=(N // BN,),
        in_specs=[
            pl.BlockSpec((NC, BN, D), lambda i: (0, i, 0)),
            pl.BlockSpec((BN, D), lambda i: (i, 0)),
            pl.BlockSpec((BN, 1), lambda i: (i, 0)),
            pl.BlockSpec((1, D), lambda i: (0, 0)),
            pl.BlockSpec((D, CP), lambda i: (0, 0)),
        ],
        out_specs=pl.BlockSpec((BN, CP), lambda i: (i, 0)),
        out_shape=jax.ShapeDtypeStruct((N, CP), f32),
    )(acc1, hs, dinv, b1r, W2p)

    acc2 = _edge_kernel_deep(nacc, cpt0s, cpt1s, rpt, CP, tc_tiling=False)(
        hs2, src_p, dst_p, zerC)

    def fin_body(a_ref, hs_ref, dv_ref, b2_ref, out_ref):
        out_ref[...] = (a_ref[0] + a_ref[1] + hs_ref[...]) * dv_ref[...] + b2_ref[...]

    outp = pl.pallas_call(
        fin_body,
        grid=(N // BN,),
        in_specs=[
            pl.BlockSpec((NC, BN, CP), lambda i: (0, i, 0)),
            pl.BlockSpec((BN, CP), lambda i: (i, 0)),
            pl.BlockSpec((BN, 1), lambda i: (i, 0)),
            pl.BlockSpec((1, CP), lambda i: (0, 0)),
        ],
        out_specs=pl.BlockSpec((BN, CP), lambda i: (i, 0)),
        out_shape=jax.ShapeDtypeStruct((N, CP), f32),
    )(acc2, hs2, dinv, b2p)

    return outp[:, :C]


# speedup vs baseline: 1.0746x; 1.0137x over previous
import functools

import jax
import jax.numpy as jnp
from jax import lax
from jax.experimental import pallas as pl
from jax.experimental.pallas import tpu as pltpu
from jax.experimental.pallas import tpu_sc as plsc

NC = 2
NS = 16
NW = NC * NS
K = 128
FRAC0 = 0.72


def _mesh():
    return plsc.VectorSubcoreMesh(core_axis_name="c", subcore_axis_name="s")


def _deg_kernel(nacc, cpt0, cpt1, rpt):
    cpt_max = max(cpt0, cpt1)

    @functools.partial(
        pl.kernel,
        out_type=jax.ShapeDtypeStruct((NC, nacc), jnp.float32),
        mesh=_mesh(),
        scratch_types=[
            pltpu.VMEM((cpt_max, 1, K), jnp.int32),
            pltpu.VMEM((K,), jnp.float32),
            pltpu.VMEM_SHARED((nacc,), jnp.float32),
            pltpu.SemaphoreType.DMA,
        ],
    )
    def deg(dst_hbm, ones_hbm, zer_hbm, out_hbm, dst_v, ones_v, deg_sh, ssem):
        cid = lax.axis_index("c")
        sid = lax.axis_index("s")
        nch = jnp.where(cid == 0, cpt0, cpt1)
        c0 = jnp.where(cid == 0, sid * cpt0, NS * cpt0 + sid * cpt1)
        r0 = sid * rpt

        @pl.when(cid == 0)
        def _():
            pltpu.sync_copy(dst_hbm.at[pl.ds(sid * cpt0, cpt0)],
                            dst_v.at[pl.ds(0, cpt0)])

        @pl.when(cid == 1)
        def _():
            pltpu.sync_copy(dst_hbm.at[pl.ds(NS * cpt0 + sid * cpt1, cpt1)],
                            dst_v.at[pl.ds(0, cpt1)])

        pltpu.sync_copy(ones_hbm, ones_v)
        pltpu.sync_copy(zer_hbm.at[pl.ds(r0, rpt)], deg_sh.at[pl.ds(r0, rpt)])
        plsc.subcore_barrier()

        def drain(j, carry):
            pltpu.make_async_copy(
                ones_v, deg_sh.at[dst_v.at[j, 0]], ssem).wait()
            return carry

        def body(j, carry):
            @pl.when(j >= 8)
            def _():
                drain(j - 8, 0)

            pltpu.async_copy(ones_v, deg_sh.at[dst_v.at[j, 0]], ssem, add=True)
            return carry

        lax.fori_loop(0, nch, body, 0)
        lax.fori_loop(jnp.maximum(nch - 8, 0), nch, drain, 0)
        plsc.subcore_barrier()
        pltpu.sync_copy(deg_sh.at[pl.ds(r0, rpt)],
                        out_hbm.at[cid, pl.ds(r0, rpt)])

    return deg


def _edge_kernel(nacc, cpt0, cpt1, rpt, F, tc_tiling=True):
    cpt_max = max(cpt0, cpt1)

    @functools.partial(
        pl.kernel,
        out_type=jax.ShapeDtypeStruct((NC, nacc, F), jnp.float32),
        mesh=_mesh(),
        compiler_params=pltpu.CompilerParams(use_tc_tiling_on_sc=tc_tiling),
        scratch_types=[
            pltpu.VMEM((1, K), jnp.int32),
            pltpu.VMEM((1, K), jnp.int32),
            pltpu.VMEM((cpt_max, 1, K), jnp.int32),
            pltpu.VMEM((K, F), jnp.float32),
            pltpu.VMEM((K, F), jnp.float32),
            pltpu.VMEM_SHARED((nacc, F), jnp.float32),
            pltpu.SemaphoreType.DMA,
            pltpu.SemaphoreType.DMA,
            pltpu.SemaphoreType.DMA,
        ],
    )
    def edge(tab_hbm, src_hbm, dst_hbm, zer_hbm, out_hbm,
             sia, sib, dst_v, rows_a, rows_b, acc_sh, gsem, ssem, isem):
        cid = lax.axis_index("c")
        sid = lax.axis_index("s")
        nch = jnp.where(cid == 0, cpt0, cpt1)
        c0 = jnp.where(cid == 0, sid * cpt0, NS * cpt0 + sid * cpt1)
        r0 = sid * rpt

        @pl.when(cid == 0)
        def _():
            pltpu.sync_copy(dst_hbm.at[pl.ds(sid * cpt0, cpt0)],
                            dst_v.at[pl.ds(0, cpt0)])

        @pl.when(cid == 1)
        def _():
            pltpu.sync_copy(dst_hbm.at[pl.ds(NS * cpt0 + sid * cpt1, cpt1)],
                            dst_v.at[pl.ds(0, cpt1)])

        pltpu.sync_copy(src_hbm.at[c0], sia)
        pltpu.async_copy(tab_hbm.at[sia.at[0]], rows_a, gsem)

        @pl.when(nch > 1)
        def _():
            pltpu.async_copy(src_hbm.at[c0 + 1], sib, isem)

        pltpu.sync_copy(zer_hbm.at[pl.ds(r0, rpt)], acc_sh.at[pl.ds(r0, rpt)])
        plsc.subcore_barrier()

        def do_iter(j, s_cur, s_nxt, rows_cur, rows_nxt):
            @pl.when(j >= 1)
            def _():
                pltpu.make_async_copy(
                    rows_nxt, acc_sh.at[dst_v.at[j - 1, 0]], ssem).wait()

            @pl.when(j + 1 < nch)
            def _():
                pltpu.make_async_copy(
                    src_hbm.at[c0 + j + 1], s_nxt, isem).wait()
                pltpu.async_copy(tab_hbm.at[s_nxt.at[0]], rows_nxt, gsem)

            pltpu.make_async_copy(
                tab_hbm.at[s_cur.at[0]], rows_cur, gsem).wait()

            @pl.when(j + 2 < nch)
            def _():
                pltpu.async_copy(src_hbm.at[c0 + j + 2], s_cur, isem)

            pltpu.async_copy(rows_cur, acc_sh.at[dst_v.at[j, 0]], ssem, add=True)

        def body(j, carry):
            par = lax.rem(j, 2)

            @pl.when(par == 0)
            def _():
                do_iter(j, sia, sib, rows_a, rows_b)

            @pl.when(par == 1)
            def _():
                do_iter(j, sib, sia, rows_b, rows_a)

            return carry

        lax.fori_loop(0, nch, body, 0)
        last = nch - 1

        @pl.when(lax.rem(last, 2) == 0)
        def _():
            pltpu.make_async_copy(rows_a, acc_sh.at[dst_v.at[last, 0]], ssem).wait()

        @pl.when(lax.rem(last, 2) == 1)
        def _():
            pltpu.make_async_copy(rows_b, acc_sh.at[dst_v.at[last, 0]], ssem).wait()

        plsc.subcore_barrier()
        pltpu.sync_copy(acc_sh.at[pl.ds(r0, rpt)],
                        out_hbm.at[cid, pl.ds(r0, rpt)])

    return edge


def _edge_kernel_deep(nacc, cpt0, cpt1, rpt, F, tc_tiling=True):
    cpt_max = max(cpt0, cpt1)
    DP = 4

    @functools.partial(
        pl.kernel,
        out_type=jax.ShapeDtypeStruct((NC, nacc, F), jnp.float32),
        mesh=_mesh(),
        compiler_params=pltpu.CompilerParams(use_tc_tiling_on_sc=tc_tiling),
        scratch_types=(
            [pltpu.VMEM((1, K), jnp.int32) for _ in range(DP)]
            + [pltpu.VMEM((K, F), jnp.float32) for _ in range(DP)]
            + [
                pltpu.VMEM((cpt_max, 1, K), jnp.int32),
                pltpu.VMEM_SHARED((nacc, F), jnp.float32),
                pltpu.SemaphoreType.DMA,
                pltpu.SemaphoreType.DMA,
                pltpu.SemaphoreType.DMA,
            ]
        ),
    )
    def edge(tab_hbm, src_hbm, dst_hbm, zer_hbm, out_hbm,
             si0, si1, si2, si3, ro0, ro1, ro2, ro3,
             dst_v, acc_sh, gsem, ssem, isem):
        sis = [si0, si1, si2, si3]
        ros = [ro0, ro1, ro2, ro3]
        cid = lax.axis_index("c")
        sid = lax.axis_index("s")
        nch = jnp.where(cid == 0, cpt0, cpt1)
        c0 = jnp.where(cid == 0, sid * cpt0, NS * cpt0 + sid * cpt1)
        r0 = sid * rpt

        @pl.when(cid == 0)
        def _():
            pltpu.sync_copy(dst_hbm.at[pl.ds(sid * cpt0, cpt0)],
                            dst_v.at[pl.ds(0, cpt0)])

        @pl.when(cid == 1)
        def _():
            pltpu.sync_copy(dst_hbm.at[pl.ds(NS * cpt0 + sid * cpt1, cpt1)],
                            dst_v.at[pl.ds(0, cpt1)])

        pltpu.sync_copy(src_hbm.at[c0], si0)
        pltpu.async_copy(tab_hbm.at[si0.at[0]], ro0, gsem)
        for t in (1, 2):
            @pl.when(nch > t)
            def _(t=t):
                pltpu.async_copy(src_hbm.at[c0 + t], sis[t], isem)
        pltpu.sync_copy(zer_hbm.at[pl.ds(r0, rpt)], acc_sh.at[pl.ds(r0, rpt)])
        plsc.subcore_barrier()

        def do_iter(j, p):
            s_n1, r_n1 = sis[(p + 1) % DP], ros[(p + 1) % DP]

            @pl.when(j >= 3)
            def _():
                pltpu.make_async_copy(
                    r_n1, acc_sh.at[dst_v.at[j - 3, 0]], ssem).wait()

            @pl.when(j + 1 < nch)
            def _():
                pltpu.make_async_copy(
                    src_hbm.at[c0 + j + 1], s_n1, isem).wait()
                pltpu.async_copy(tab_hbm.at[s_n1.at[0]], r_n1, gsem)

            @pl.when(j + 3 < nch)
            def _():
                pltpu.async_copy(src_hbm.at[c0 + j + 3], sis[(p + 3) % DP], isem)

            pltpu.make_async_copy(tab_hbm.at[sis[p].at[0]], ros[p], gsem).wait()
            pltpu.async_copy(ros[p], acc_sh.at[dst_v.at[j, 0]], ssem, add=True)

        def body(j, carry):
            par = lax.rem(j, DP)
            for p in range(DP):
                @pl.when(par == p)
                def _(p=p):
                    do_iter(j, p)
            return carry

        lax.fori_loop(0, nch, body, 0)

        def drain(t):
            for p in range(DP):
                @pl.when((t >= 0) & (lax.rem(t, DP) == p))
                def _(p=p):
                    pltpu.make_async_copy(
                        ros[p], acc_sh.at[dst_v.at[t, 0]], ssem).wait()

        drain(nch - 3)
        drain(nch - 2)
        drain(nch - 1)
        plsc.subcore_barrier()
        pltpu.sync_copy(acc_sh.at[pl.ds(r0, rpt)],
                        out_hbm.at[cid, pl.ds(r0, rpt)])

    return edge


def _pick_bn(n):
    for bn in (2000, 1024, 512, 400, 256, 200, 128, 80, 64, 40, 32, 16, 8):
        if n % bn == 0:
            return bn
    return 1


def _flat_chunks(arr, pad_val, tot_ch, n_real):
    flat = jnp.concatenate(
        [arr, jnp.full((tot_ch * K - n_real,), pad_val, jnp.int32)])
    return flat.reshape(tot_ch, 1, K)


def kernel(x, edge_index, W1, b1, W2, b2):
    N, D = x.shape
    E = edge_index.shape[1]
    C = W2.shape[1]
    CP = 16

    nch = -(-E // K)
    cpt0 = max(1, round(nch * FRAC0 / NS))
    cpt1 = -(-(nch - cpt0 * NS) // NS)
    cpt0s = max(1, round(nch * 0.5 / NS))
    cpt1s = -(-(nch - cpt0s * NS) // NS)
    tot_ch = max((cpt0 + cpt1) * NS, (cpt0s + cpt1s) * NS)
    nacc = -(-(N + 1) // 128) * 128
    rpt = nacc // NS
    nacc_d = -(-(N + 1) // (NS * 256)) * (NS * 256)
    rpt_d = nacc_d // NS
    BN = _pick_bn(N)

    f32 = jnp.float32
    src_p = _flat_chunks(edge_index[0], 0, tot_ch, E)
    dst_p = _flat_chunks(edge_index[1], N, tot_ch, E)
    ones_k = jnp.ones((K,), f32)
    zer1 = jnp.zeros((nacc_d,), f32)
    zerD = jnp.zeros((nacc, D), f32)
    zerC = jnp.zeros((nacc, CP), f32)
    b1r = b1.reshape(1, D)
    W2p = jnp.zeros((D, CP), f32).at[:, :C].set(W2)
    b2p = jnp.zeros((1, CP), f32).at[0, :C].set(b2)

    deg_part = _deg_kernel(nacc_d, cpt0s, cpt1s, rpt_d)(dst_p, ones_k, zer1)
    deg_t = deg_part.T[:N]

    def mm1a_body(x_ref, w_ref, h_ref):
        h_ref[...] = jnp.dot(x_ref[...], w_ref[...], preferred_element_type=f32)

    h = pl.pallas_call(
        mm1a_body,
        grid=(N // BN,),
        in_specs=[
            pl.BlockSpec((BN, D), lambda i: (i, 0)),
            pl.BlockSpec((D, D), lambda i: (0, 0)),
        ],
        out_specs=pl.BlockSpec((BN, D), lambda i: (i, 0)),
        out_shape=jax.ShapeDtypeStruct((N, D), f32),
    )(x, W1)

    def mm1b_body(h_ref, dg_ref, hs_ref, dv_ref):
        dv = lax.rsqrt(dg_ref[:, 0:1] + dg_ref[:, 1:2] + 1.0)
        hs_ref[...] = h_ref[...] * dv
        dv_ref[...] = dv

    hs, dinv = pl.pallas_call(
        mm1b_body,
        grid=(N // BN,),
        in_specs=[
            pl.BlockSpec((BN, D), lambda i: (i, 0)),
            pl.BlockSpec((BN, 2), lambda i: (i, 0)),
        ],
        out_specs=[
            pl.BlockSpec((BN, D), lambda i: (i, 0)),
            pl.BlockSpec((BN, 1), lambda i: (i, 0)),
        ],
        out_shape=[
            jax.ShapeDtypeStruct((N, D), f32),
            jax.ShapeDtypeStruct((N, 1), f32),
        ],
    )(h, deg_t)

    acc1 = _edge_kernel(nacc, cpt0, cpt1, rpt, D)(hs, src_p, dst_p, zerD)

    def mid_body(a_ref, hs_ref, dv_ref, b1_ref, w2_ref, out_ref):
        t = (a_ref[0] + a_ref[1] + hs_ref[...]) * dv_ref[...] + b1_ref[...]
        g = jnp.where(t >= 0, t, 0.01 * t)
        h2 = jnp.dot(g, w2_ref[...], preferred_element_type=f32)
        out_ref[...] = h2 * dv_ref[...]

    hs2 = pl.pallas_call(
        mid_body,
        grid=(N // BN,),
        in_specs=[
            pl.BlockSpec((NC, BN, D), lambda i: (0, i, 0)),
            pl.BlockSpec((BN, D), lambda i: (i, 0)),
            pl.BlockSpec((BN, 1), lambda i: (i, 0)),
            pl.BlockSpec((1, D), lambda i: (0, 0)),
            pl.BlockSpec((D, CP), lambda i: (0, 0)),
        ],
        out_specs=pl.BlockSpec((BN, CP), lambda i: (i, 0)),
        out_shape=jax.ShapeDtypeStruct((N, CP), f32),
    )(acc1, hs, dinv, b1r, W2p)

    acc2 = _edge_kernel_deep(nacc, cpt0s, cpt1s, rpt, CP, tc_tiling=False)(
        hs2, src_p, dst_p, zerC)

    def fin_body(a_ref, hs_ref, dv_ref, b2_ref, out_ref):
        out_ref[...] = (a_ref[0] + a_ref[1] + hs_ref[...]) * dv_ref[...] + b2_ref[...]

    outp = pl.pallas_call(
        fin_body,
        grid=(N // BN,),
        in_specs=[
            pl.BlockSpec((NC, BN, CP), lambda i: (0, i, 0)),
            pl.BlockSpec((BN, CP), lambda i: (i, 0)),
            pl.BlockSpec((BN, 1), lambda i: (i, 0)),
            pl.BlockSpec((1, CP), lambda i: (0, 0)),
        ],
        out_specs=pl.BlockSpec((BN, CP), lambda i: (i, 0)),
        out_shape=jax.ShapeDtypeStruct((N, CP), f32),
    )(acc2, hs2, dinv, b2p)

    return outp[:, :C]
